# baseline (speedup 1.0000x reference)
import jax
import jax.numpy as jnp
from jax.experimental import pallas as pl
from jax.experimental.pallas import tpu as pltpu
from jax.experimental.pallas import tpu_sc as plsc

D = 1024
E = 8
N_TOK = 4096
BLK = 256
A_TOT = N_TOK * 2
PAD = A_TOT + E * BLK
NB = PAD // BLK
GBLK = 16
EPAD = 128
YASG = 2 * N_TOK + BLK
SC_W = 32

_BIG = 1 << 30


def _gate_body(x_ref, w_ref, b_ref, i0_ref, i1_ref, w0_ref, w1_ref):
    logits = jnp.dot(x_ref[...], w_ref[...],
                     preferred_element_type=jnp.float32) + b_ref[...]
    m = jnp.max(logits, axis=1, keepdims=True)
    ex = jnp.exp(logits - m)
    p = ex / jnp.sum(ex, axis=1, keepdims=True)
    iota = jax.lax.broadcasted_iota(jnp.int32, (BLK, EPAD), 1)
    v0 = jnp.max(p, axis=1, keepdims=True)
    i0 = jnp.min(jnp.where(p == v0, iota, _BIG), axis=1, keepdims=True)
    p2 = jnp.where(iota == i0, -1.0, p)
    v1 = jnp.max(p2, axis=1, keepdims=True)
    i1 = jnp.min(jnp.where(p2 == v1, iota, _BIG), axis=1, keepdims=True)
    zeros_i = jnp.zeros((BLK, EPAD), jnp.int32)
    i0_ref[...] = i0 + zeros_i
    i1_ref[...] = i1 + zeros_i
    zeros_f = jnp.zeros((BLK, EPAD), jnp.float32)
    w0_ref[...] = v0 + zeros_f
    w1_ref[...] = v1 + zeros_f


def _gating(x, gate_W, gate_b):
    gwp = jnp.zeros((D, EPAD), jnp.float32).at[:, :E].set(gate_W)
    gbp = jnp.full((1, EPAD), -1e30, jnp.float32).at[0, :E].set(gate_b)
    out_sh = [jax.ShapeDtypeStruct((N_TOK, EPAD), jnp.int32),
              jax.ShapeDtypeStruct((N_TOK, EPAD), jnp.int32),
              jax.ShapeDtypeStruct((N_TOK, EPAD), jnp.float32),
              jax.ShapeDtypeStruct((N_TOK, EPAD), jnp.float32)]
    blk = pl.BlockSpec((BLK, EPAD), lambda i: (i, 0))
    return pl.pallas_call(
        _gate_body,
        grid=(GBLK,),
        in_specs=[pl.BlockSpec((BLK, D), lambda i: (i, 0)),
                  pl.BlockSpec((D, EPAD), lambda i: (0, 0)),
                  pl.BlockSpec((1, EPAD), lambda i: (0, 0))],
        out_specs=[blk, blk, blk, blk],
        out_shape=out_sh,
    )(x, gwp, gbp)


SPLIT = 4
DSUB = D // SPLIT
IW = 128


def _sc_gather(x, tok_for_row):
    mesh = plsc.VectorSubcoreMesh(core_axis_name="core",
                                  subcore_axis_name="subcore")
    idx4 = (tok_for_row[:, None] * SPLIT +
            jnp.arange(SPLIT, dtype=jnp.int32)[None, :]).reshape(1, PAD * SPLIT)

    @pl.kernel(out_type=jax.ShapeDtypeStruct((PAD * SPLIT, DSUB), jnp.float32),
               mesh=mesh)
    def gather_k(x_hbm, i_hbm, o_hbm):
        def body(i_vmem, o_vmem):
            pltpu.sync_copy(x_hbm.at[i_vmem.at[0]], o_vmem)

        pltpu.emit_pipeline(
            body,
            grid=(PAD * SPLIT // IW,),
            in_specs=[pl.BlockSpec((1, IW), lambda i: (0, i))],
            out_specs=[pl.BlockSpec((IW, DSUB), lambda i: (i, 0))],
            core_axis_name=("core", "subcore"),
            dimension_semantics=(pltpu.PARALLEL,),
        )(i_hbm, o_hbm)

    return gather_k(x.reshape(N_TOK * SPLIT, DSUB), idx4).reshape(PAD, D)


def _ffn_body(be_ref, x_ref, w1_ref, b1_ref, w2_ref, b2_ref, o_ref):
    h = jnp.dot(x_ref[...], w1_ref[0],
                preferred_element_type=jnp.float32) + b1_ref[0]
    h = jnp.maximum(h, 0.0)
    o_ref[...] = jnp.dot(h, w2_ref[0],
                         preferred_element_type=jnp.float32) + b2_ref[0]


def _grouped_ffn(block_expert, x_sorted, fc1_W, fc1_b, fc2_W, fc2_b):
    spec = pltpu.PrefetchScalarGridSpec(
        num_scalar_prefetch=1,
        grid=(NB,),
        in_specs=[
            pl.BlockSpec((BLK, D), lambda i, be: (i, 0)),
            pl.BlockSpec((1, D, D), lambda i, be: (be[i], 0, 0)),
            pl.BlockSpec((1, 1, D), lambda i, be: (be[i], 0, 0)),
            pl.BlockSpec((1, D, D), lambda i, be: (be[i], 0, 0)),
            pl.BlockSpec((1, 1, D), lambda i, be: (be[i], 0, 0)),
        ],
        out_specs=pl.BlockSpec((BLK, D), lambda i, be: (i, 0)),
    )
    return pl.pallas_call(
        _ffn_body,
        grid_spec=spec,
        out_shape=jax.ShapeDtypeStruct((PAD, D), jnp.float32),
    )(block_expert, x_sorted, fc1_W, fc1_b.reshape(E, 1, D),
      fc2_W, fc2_b.reshape(E, 1, D))


def _sc_scatter(y_sorted, dest):
    mesh = plsc.VectorSubcoreMesh(core_axis_name="core",
                                  subcore_axis_name="subcore")
    dest4 = (dest[:, None] * SPLIT +
             jnp.arange(SPLIT, dtype=jnp.int32)[None, :]).reshape(1, PAD * SPLIT)

    @pl.kernel(out_type=jax.ShapeDtypeStruct((YASG * SPLIT, DSUB), jnp.float32),
               mesh=mesh)
    def scatter_k(y_hbm, d_hbm, o_hbm):
        def body(y_vmem, i_vmem):
            pltpu.sync_copy(y_vmem, o_hbm.at[i_vmem.at[0]])

        pltpu.emit_pipeline(
            body,
            grid=(PAD * SPLIT // IW,),
            in_specs=[pl.BlockSpec((IW, DSUB), lambda i: (i, 0)),
                      pl.BlockSpec((1, IW), lambda i: (0, i))],
            out_specs=[],
            core_axis_name=("core", "subcore"),
            dimension_semantics=(pltpu.PARALLEL,),
        )(y_hbm, d_hbm)

    return scatter_k(y_sorted.reshape(PAD * SPLIT, DSUB),
                     dest4).reshape(YASG, D)


def _combine_body(y0_ref, y1_ref, w0_ref, w1_ref, o_ref):
    o_ref[...] = (w0_ref[:, 0:1] * y0_ref[...] +
                  w1_ref[:, 0:1] * y1_ref[...])


def _combine(yasg, w0b, w1b):
    wblk = pl.BlockSpec((BLK, EPAD), lambda i: (i, 0))
    return pl.pallas_call(
        _combine_body,
        grid=(GBLK,),
        in_specs=[pl.BlockSpec((BLK, D), lambda i: (i, 0)),
                  pl.BlockSpec((BLK, D), lambda i: (i + GBLK, 0)),
                  wblk, wblk],
        out_specs=pl.BlockSpec((BLK, D), lambda i: (i, 0)),
        out_shape=jax.ShapeDtypeStruct((N_TOK, D), jnp.float32),
    )(yasg, yasg, w0b, w1b)


def kernel(x, gate_W, gate_b, fc1_W, fc1_b, fc2_W, fc2_b):
    i0b, i1b, w0b, w1b = _gating(x, gate_W, gate_b)
    i0 = i0b[:, 0]
    i1 = i1b[:, 0]

    flat_e = jnp.stack([i0, i1], axis=1).reshape(-1)
    onehot = (flat_e[:, None] == jnp.arange(E)[None, :]).astype(jnp.int32)
    csum = jnp.cumsum(onehot, axis=0)
    counts = csum[-1]
    rank = jnp.take_along_axis(csum, flat_e[:, None], 1)[:, 0] - 1
    pcounts = ((counts + BLK - 1) // BLK) * BLK
    poff = jnp.concatenate([jnp.zeros((1,), pcounts.dtype),
                            jnp.cumsum(pcounts)[:-1]])
    pp = poff[flat_e] + rank
    a = jnp.arange(A_TOT)
    tok_for_row = jnp.zeros((PAD,), jnp.int32).at[pp].set(
        (a // 2).astype(jnp.int32))
    dest = (2 * N_TOK + (jnp.arange(PAD) % BLK)).astype(jnp.int32)
    dest = dest.at[pp].set(((a % 2) * N_TOK + a // 2).astype(jnp.int32))
    pend = jnp.cumsum(pcounts)
    starts = jnp.arange(NB) * BLK
    block_expert = jnp.minimum(
        jnp.sum((starts[:, None] >= pend[None, :]).astype(jnp.int32), axis=1),
        E - 1).astype(jnp.int32)

    x_sorted = _sc_gather(x, tok_for_row)
    y_sorted = _grouped_ffn(block_expert, x_sorted, fc1_W, fc1_b, fc2_W, fc2_b)
    yasg = _sc_scatter(y_sorted, dest)
    return _combine(yasg, w0b, w1b)


# speedup vs baseline: 1.5334x; 1.5334x over previous
import jax
import jax.numpy as jnp
from jax.experimental import pallas as pl
from jax.experimental.pallas import tpu as pltpu
from jax.experimental.pallas import tpu_sc as plsc

D = 1024
E = 8
N_TOK = 4096
BLK = 128
A_TOT = N_TOK * 2
PAD = A_TOT + E * BLK
NB = PAD // BLK
TBLK = 256
GBLK = N_TOK // TBLK
EPAD = 128
YASG = 2 * N_TOK + TBLK
NW = 32
RPW = PAD // NW
CH = 48
NCH = RPW // CH

_BIG = 1 << 30


def _gate_body(x_ref, w_ref, b_ref, i0_ref, i1_ref, w0_ref, w1_ref):
    logits = jnp.dot(x_ref[...], w_ref[...],
                     preferred_element_type=jnp.float32) + b_ref[...]
    m = jnp.max(logits, axis=1, keepdims=True)
    ex = jnp.exp(logits - m)
    p = ex / jnp.sum(ex, axis=1, keepdims=True)
    iota = jax.lax.broadcasted_iota(jnp.int32, (TBLK, EPAD), 1)
    v0 = jnp.max(p, axis=1, keepdims=True)
    i0 = jnp.min(jnp.where(p == v0, iota, _BIG), axis=1, keepdims=True)
    p2 = jnp.where(iota == i0, -1.0, p)
    v1 = jnp.max(p2, axis=1, keepdims=True)
    i1 = jnp.min(jnp.where(p2 == v1, iota, _BIG), axis=1, keepdims=True)
    zeros_i = jnp.zeros((TBLK, EPAD), jnp.int32)
    i0_ref[...] = i0 + zeros_i
    i1_ref[...] = i1 + zeros_i
    zeros_f = jnp.zeros((TBLK, EPAD), jnp.float32)
    w0_ref[...] = v0 + zeros_f
    w1_ref[...] = v1 + zeros_f


def _gating(x, gate_W, gate_b):
    gwp = jnp.zeros((D, EPAD), jnp.float32).at[:, :E].set(gate_W)
    gbp = jnp.full((1, EPAD), -1e30, jnp.float32).at[0, :E].set(gate_b)
    out_sh = [jax.ShapeDtypeStruct((N_TOK, EPAD), jnp.int32),
              jax.ShapeDtypeStruct((N_TOK, EPAD), jnp.int32),
              jax.ShapeDtypeStruct((N_TOK, EPAD), jnp.float32),
              jax.ShapeDtypeStruct((N_TOK, EPAD), jnp.float32)]
    blk = pl.BlockSpec((TBLK, EPAD), lambda i: (i, 0))
    return pl.pallas_call(
        _gate_body,
        grid=(GBLK,),
        in_specs=[pl.BlockSpec((TBLK, D), lambda i: (i, 0)),
                  pl.BlockSpec((D, EPAD), lambda i: (0, 0)),
                  pl.BlockSpec((1, EPAD), lambda i: (0, 0))],
        out_specs=[blk, blk, blk, blk],
        out_shape=out_sh,
    )(x, gwp, gbp)


def _sc_gather(x, tok_for_row):
    mesh = plsc.VectorSubcoreMesh(core_axis_name="core",
                                  subcore_axis_name="subcore")

    @pl.kernel(out_type=jax.ShapeDtypeStruct((PAD, D), jnp.float32),
               mesh=mesh,
               scratch_types=[pltpu.VMEM((CH,), jnp.int32),
                              pltpu.VMEM((CH,), jnp.int32),
                              pltpu.VMEM((CH, D), jnp.float32),
                              pltpu.VMEM((CH, D), jnp.float32),
                              pltpu.SemaphoreType.DMA,
                              pltpu.SemaphoreType.DMA])
    def gather_k(x_hbm, tok_hbm, o_hbm, idx0, idx1, buf0, buf1, sem0, sem1):
        wid = (jax.lax.axis_index("subcore") * 2
               + jax.lax.axis_index("core"))
        base = wid * RPW
        idx = (idx0, idx1)
        buf = (buf0, buf1)
        sem = (sem0, sem1)
        cps = [None, None]
        pltpu.sync_copy(tok_hbm.at[pl.ds(base, CH)], idx0)
        cps[0] = pltpu.async_copy(x_hbm.at[idx0], buf0, sem0)
        for c in range(1, NCH):
            s = c % 2
            pltpu.sync_copy(tok_hbm.at[pl.ds(base + c * CH, CH)], idx[s])
            cps[s] = pltpu.async_copy(x_hbm.at[idx[s]], buf[s], sem[s])
            cps[1 - s].wait()
            pltpu.sync_copy(buf[1 - s],
                            o_hbm.at[pl.ds(base + (c - 1) * CH, CH)])
        last = (NCH - 1) % 2
        cps[last].wait()
        pltpu.sync_copy(buf[last], o_hbm.at[pl.ds(base + (NCH - 1) * CH, CH)])

    return gather_k(x, tok_for_row)


def _ffn_body(be_ref, x_ref, w1_ref, b1_ref, w2_ref, b2_ref, o_ref):
    h = jnp.dot(x_ref[...], w1_ref[0],
                preferred_element_type=jnp.float32) + b1_ref[0]
    h = jnp.maximum(h, 0.0)
    o_ref[...] = jnp.dot(h, w2_ref[0],
                         preferred_element_type=jnp.float32) + b2_ref[0]


def _grouped_ffn(block_expert, x_sorted, fc1_W, fc1_b, fc2_W, fc2_b):
    spec = pltpu.PrefetchScalarGridSpec(
        num_scalar_prefetch=1,
        grid=(NB,),
        in_specs=[
            pl.BlockSpec((BLK, D), lambda i, be: (i, 0)),
            pl.BlockSpec((1, D, D), lambda i, be: (be[i], 0, 0)),
            pl.BlockSpec((1, 1, D), lambda i, be: (be[i], 0, 0)),
            pl.BlockSpec((1, D, D), lambda i, be: (be[i], 0, 0)),
            pl.BlockSpec((1, 1, D), lambda i, be: (be[i], 0, 0)),
        ],
        out_specs=pl.BlockSpec((BLK, D), lambda i, be: (i, 0)),
    )
    return pl.pallas_call(
        _ffn_body,
        grid_spec=spec,
        out_shape=jax.ShapeDtypeStruct((PAD, D), jnp.float32),
    )(block_expert, x_sorted, fc1_W, fc1_b.reshape(E, 1, D),
      fc2_W, fc2_b.reshape(E, 1, D))


def _sc_scatter(y_sorted, dest):
    mesh = plsc.VectorSubcoreMesh(core_axis_name="core",
                                  subcore_axis_name="subcore")

    @pl.kernel(out_type=jax.ShapeDtypeStruct((YASG, D), jnp.float32),
               mesh=mesh,
               scratch_types=[pltpu.VMEM((CH,), jnp.int32),
                              pltpu.VMEM((CH,), jnp.int32),
                              pltpu.VMEM((CH, D), jnp.float32),
                              pltpu.VMEM((CH, D), jnp.float32),
                              pltpu.SemaphoreType.DMA,
                              pltpu.SemaphoreType.DMA])
    def scatter_k(y_hbm, d_hbm, o_hbm, idx0, idx1, buf0, buf1, sem0, sem1):
        wid = (jax.lax.axis_index("subcore") * 2
               + jax.lax.axis_index("core"))
        base = wid * RPW
        idx = (idx0, idx1)
        buf = (buf0, buf1)
        sem = (sem0, sem1)
        cps = [None, None]
        for c in range(NCH):
            s = c % 2
            if cps[s] is not None:
                cps[s].wait()
            pltpu.sync_copy(y_hbm.at[pl.ds(base + c * CH, CH)], buf[s])
            pltpu.sync_copy(d_hbm.at[pl.ds(base + c * CH, CH)], idx[s])
            cps[s] = pltpu.async_copy(buf[s], o_hbm.at[idx[s]], sem[s])
        cps[0].wait()
        cps[1].wait()

    return scatter_k(y_sorted, dest)


def _combine_body(y0_ref, y1_ref, w0_ref, w1_ref, o_ref):
    o_ref[...] = (w0_ref[:, 0:1] * y0_ref[...] +
                  w1_ref[:, 0:1] * y1_ref[...])


def _combine(yasg, w0b, w1b):
    wblk = pl.BlockSpec((TBLK, EPAD), lambda i: (i, 0))
    return pl.pallas_call(
        _combine_body,
        grid=(GBLK,),
        in_specs=[pl.BlockSpec((TBLK, D), lambda i: (i, 0)),
                  pl.BlockSpec((TBLK, D), lambda i: (i + GBLK, 0)),
                  wblk, wblk],
        out_specs=pl.BlockSpec((TBLK, D), lambda i: (i, 0)),
        out_shape=jax.ShapeDtypeStruct((N_TOK, D), jnp.float32),
    )(yasg, yasg, w0b, w1b)


def kernel(x, gate_W, gate_b, fc1_W, fc1_b, fc2_W, fc2_b):
    i0b, i1b, w0b, w1b = _gating(x, gate_W, gate_b)
    i0 = i0b[:, 0]
    i1 = i1b[:, 0]

    flat_e = jnp.stack([i0, i1], axis=1).reshape(-1)
    onehot = (flat_e[:, None] == jnp.arange(E)[None, :]).astype(jnp.int32)
    csum = jnp.cumsum(onehot, axis=0)
    counts = csum[-1]
    rank = jnp.take_along_axis(csum, flat_e[:, None], 1)[:, 0] - 1
    pcounts = ((counts + BLK - 1) // BLK) * BLK
    poff = jnp.concatenate([jnp.zeros((1,), pcounts.dtype),
                            jnp.cumsum(pcounts)[:-1]])
    pp = poff[flat_e] + rank
    a = jnp.arange(A_TOT)
    tok_for_row = jnp.zeros((PAD,), jnp.int32).at[pp].set(
        (a // 2).astype(jnp.int32))
    dest = (2 * N_TOK + (jnp.arange(PAD) % TBLK)).astype(jnp.int32)
    dest = dest.at[pp].set(((a % 2) * N_TOK + a // 2).astype(jnp.int32))
    pend = jnp.cumsum(pcounts)
    starts = jnp.arange(NB) * BLK
    block_expert = jnp.minimum(
        jnp.sum((starts[:, None] >= pend[None, :]).astype(jnp.int32), axis=1),
        E - 1).astype(jnp.int32)

    x_sorted = _sc_gather(x, tok_for_row)
    y_sorted = _grouped_ffn(block_expert, x_sorted, fc1_W, fc1_b, fc2_W, fc2_b)
    yasg = _sc_scatter(y_sorted, dest)
    return _combine(yasg, w0b, w1b)


# speedup vs baseline: 2.0053x; 1.3078x over previous
import jax
import jax.numpy as jnp
from jax.experimental import pallas as pl
from jax.experimental.pallas import tpu as pltpu
from jax.experimental.pallas import tpu_sc as plsc

D = 1024
E = 8
N_TOK = 4096
BLK = 128
A_TOT = N_TOK * 2
PAD = A_TOT + E * BLK
NB = PAD // BLK
TBLK = 256
GBLK = N_TOK // TBLK
EPAD = 128
YASG = 2 * N_TOK + TBLK
NW = 32
RPW = PAD // NW
CH = 48
NCH = RPW // CH

_BIG = 1 << 30


def _gate_body(x_ref, w_ref, b_ref, i0_ref, i1_ref, w0_ref, w1_ref):
    logits = jnp.dot(x_ref[...], w_ref[...],
                     preferred_element_type=jnp.float32) + b_ref[...]
    m = jnp.max(logits, axis=1, keepdims=True)
    ex = jnp.exp(logits - m)
    p = ex / jnp.sum(ex, axis=1, keepdims=True)
    iota = jax.lax.broadcasted_iota(jnp.int32, (TBLK, EPAD), 1)
    v0 = jnp.max(p, axis=1, keepdims=True)
    i0 = jnp.min(jnp.where(p == v0, iota, _BIG), axis=1, keepdims=True)
    p2 = jnp.where(iota == i0, -1.0, p)
    v1 = jnp.max(p2, axis=1, keepdims=True)
    i1 = jnp.min(jnp.where(p2 == v1, iota, _BIG), axis=1, keepdims=True)
    zeros_i = jnp.zeros((TBLK, EPAD), jnp.int32)
    i0_ref[...] = i0 + zeros_i
    i1_ref[...] = i1 + zeros_i
    zeros_f = jnp.zeros((TBLK, EPAD), jnp.float32)
    w0_ref[...] = v0 + zeros_f
    w1_ref[...] = v1 + zeros_f


def _gating(x, gate_W, gate_b):
    gwp = jnp.zeros((D, EPAD), jnp.float32).at[:, :E].set(gate_W)
    gbp = jnp.full((1, EPAD), -1e30, jnp.float32).at[0, :E].set(gate_b)
    out_sh = [jax.ShapeDtypeStruct((N_TOK, EPAD), jnp.int32),
              jax.ShapeDtypeStruct((N_TOK, EPAD), jnp.int32),
              jax.ShapeDtypeStruct((N_TOK, EPAD), jnp.float32),
              jax.ShapeDtypeStruct((N_TOK, EPAD), jnp.float32)]
    blk = pl.BlockSpec((TBLK, EPAD), lambda i: (i, 0))
    return pl.pallas_call(
        _gate_body,
        grid=(GBLK,),
        in_specs=[pl.BlockSpec((TBLK, D), lambda i: (i, 0)),
                  pl.BlockSpec((D, EPAD), lambda i: (0, 0)),
                  pl.BlockSpec((1, EPAD), lambda i: (0, 0))],
        out_specs=[blk, blk, blk, blk],
        out_shape=out_sh,
    )(x, gwp, gbp)


TPW = N_TOK // NW
DCH = 32
DNCH = TPW // DCH


def _sc_dispatch(x, pos0, pos1):
    mesh = plsc.VectorSubcoreMesh(core_axis_name="core",
                                  subcore_axis_name="subcore")

    @pl.kernel(out_type=jax.ShapeDtypeStruct((PAD, D), jnp.float32),
               mesh=mesh,
               scratch_types=[pltpu.VMEM((DCH,), jnp.int32),
                              pltpu.VMEM((DCH,), jnp.int32),
                              pltpu.VMEM((DCH,), jnp.int32),
                              pltpu.VMEM((DCH,), jnp.int32),
                              pltpu.VMEM((DCH, D), jnp.float32),
                              pltpu.VMEM((DCH, D), jnp.float32),
                              pltpu.SemaphoreType.DMA,
                              pltpu.SemaphoreType.DMA,
                              pltpu.SemaphoreType.DMA,
                              pltpu.SemaphoreType.DMA])
    def dispatch_k(x_hbm, p0_hbm, p1_hbm, o_hbm,
                   ia0, ia1, ib0, ib1, buf0, buf1, sa0, sa1, sb0, sb1):
        wid = (jax.lax.axis_index("subcore") * 2
               + jax.lax.axis_index("core"))
        base = wid * TPW
        ia = (ia0, ia1)
        ib = (ib0, ib1)
        buf = (buf0, buf1)
        sa = (sa0, sa1)
        sb = (sb0, sb1)
        cpa = [None, None]
        cpb = [None, None]
        for c in range(DNCH):
            s = c % 2
            if cpa[s] is not None:
                cpa[s].wait()
                cpb[s].wait()
            off = base + c * DCH
            pltpu.sync_copy(x_hbm.at[pl.ds(off, DCH)], buf[s])
            pltpu.sync_copy(p0_hbm.at[pl.ds(off, DCH)], ia[s])
            pltpu.sync_copy(p1_hbm.at[pl.ds(off, DCH)], ib[s])
            cpa[s] = pltpu.async_copy(buf[s], o_hbm.at[ia[s]], sa[s])
            cpb[s] = pltpu.async_copy(buf[s], o_hbm.at[ib[s]], sb[s])
        cpa[0].wait()
        cpb[0].wait()
        cpa[1].wait()
        cpb[1].wait()

    return dispatch_k(x, pos0, pos1)


def _ffn_body(be_ref, x_ref, w1_ref, b1_ref, w2_ref, b2_ref, o_ref):
    h = jnp.dot(x_ref[...], w1_ref[0],
                preferred_element_type=jnp.float32) + b1_ref[0]
    h = jnp.maximum(h, 0.0)
    o_ref[...] = jnp.dot(h, w2_ref[0],
                         preferred_element_type=jnp.float32) + b2_ref[0]


def _grouped_ffn(block_expert, x_sorted, fc1_W, fc1_b, fc2_W, fc2_b):
    spec = pltpu.PrefetchScalarGridSpec(
        num_scalar_prefetch=1,
        grid=(NB,),
        in_specs=[
            pl.BlockSpec((BLK, D), lambda i, be: (i, 0)),
            pl.BlockSpec((1, D, D), lambda i, be: (be[i], 0, 0)),
            pl.BlockSpec((1, 1, D), lambda i, be: (be[i], 0, 0)),
            pl.BlockSpec((1, D, D), lambda i, be: (be[i], 0, 0)),
            pl.BlockSpec((1, 1, D), lambda i, be: (be[i], 0, 0)),
        ],
        out_specs=pl.BlockSpec((BLK, D), lambda i, be: (i, 0)),
    )
    return pl.pallas_call(
        _ffn_body,
        grid_spec=spec,
        out_shape=jax.ShapeDtypeStruct((PAD, D), jnp.float32),
    )(block_expert, x_sorted, fc1_W, fc1_b.reshape(E, 1, D),
      fc2_W, fc2_b.reshape(E, 1, D))


def _sc_scatter(y_sorted, dest):
    mesh = plsc.VectorSubcoreMesh(core_axis_name="core",
                                  subcore_axis_name="subcore")

    @pl.kernel(out_type=jax.ShapeDtypeStruct((YASG, D), jnp.float32),
               mesh=mesh,
               scratch_types=[pltpu.VMEM((CH,), jnp.int32),
                              pltpu.VMEM((CH,), jnp.int32),
                              pltpu.VMEM((CH, D), jnp.float32),
                              pltpu.VMEM((CH, D), jnp.float32),
                              pltpu.SemaphoreType.DMA,
                              pltpu.SemaphoreType.DMA])
    def scatter_k(y_hbm, d_hbm, o_hbm, idx0, idx1, buf0, buf1, sem0, sem1):
        wid = (jax.lax.axis_index("subcore") * 2
               + jax.lax.axis_index("core"))
        base = wid * RPW
        idx = (idx0, idx1)
        buf = (buf0, buf1)
        sem = (sem0, sem1)
        cps = [None, None]
        for c in range(NCH):
            s = c % 2
            if cps[s] is not None:
                cps[s].wait()
            pltpu.sync_copy(y_hbm.at[pl.ds(base + c * CH, CH)], buf[s])
            pltpu.sync_copy(d_hbm.at[pl.ds(base + c * CH, CH)], idx[s])
            cps[s] = pltpu.async_copy(buf[s], o_hbm.at[idx[s]], sem[s])
        cps[0].wait()
        cps[1].wait()

    return scatter_k(y_sorted, dest)


def _combine_body(y0_ref, y1_ref, w0_ref, w1_ref, o_ref):
    o_ref[...] = (w0_ref[:, 0:1] * y0_ref[...] +
                  w1_ref[:, 0:1] * y1_ref[...])


def _combine(yasg, w0b, w1b):
    wblk = pl.BlockSpec((TBLK, EPAD), lambda i: (i, 0))
    return pl.pallas_call(
        _combine_body,
        grid=(GBLK,),
        in_specs=[pl.BlockSpec((TBLK, D), lambda i: (i, 0)),
                  pl.BlockSpec((TBLK, D), lambda i: (i + GBLK, 0)),
                  wblk, wblk],
        out_specs=pl.BlockSpec((TBLK, D), lambda i: (i, 0)),
        out_shape=jax.ShapeDtypeStruct((N_TOK, D), jnp.float32),
    )(yasg, yasg, w0b, w1b)


def kernel(x, gate_W, gate_b, fc1_W, fc1_b, fc2_W, fc2_b):
    i0b, i1b, w0b, w1b = _gating(x, gate_W, gate_b)
    i0 = i0b[:, 0]
    i1 = i1b[:, 0]

    flat_e = jnp.stack([i0, i1], axis=1).reshape(-1)
    onehot = (flat_e[:, None] == jnp.arange(E)[None, :]).astype(jnp.int32)
    csum = jnp.cumsum(onehot, axis=0)
    counts = csum[-1]
    rank = jnp.take_along_axis(csum, flat_e[:, None], 1)[:, 0] - 1
    pcounts = ((counts + BLK - 1) // BLK) * BLK
    poff = jnp.concatenate([jnp.zeros((1,), pcounts.dtype),
                            jnp.cumsum(pcounts)[:-1]])
    pp = poff[flat_e] + rank
    a = jnp.arange(A_TOT)
    pos = pp.reshape(N_TOK, 2).astype(jnp.int32)
    dest = (2 * N_TOK + (jnp.arange(PAD) % TBLK)).astype(jnp.int32)
    dest = dest.at[pp].set(((a % 2) * N_TOK + a // 2).astype(jnp.int32))
    pend = jnp.cumsum(pcounts)
    starts = jnp.arange(NB) * BLK
    block_expert = jnp.minimum(
        jnp.sum((starts[:, None] >= pend[None, :]).astype(jnp.int32), axis=1),
        E - 1).astype(jnp.int32)

    x_sorted = _sc_dispatch(x, pos[:, 0], pos[:, 1])
    y_sorted = _grouped_ffn(block_expert, x_sorted, fc1_W, fc1_b, fc2_W, fc2_b)
    yasg = _sc_scatter(y_sorted, dest)
    return _combine(yasg, w0b, w1b)


# speedup vs baseline: 2.0143x; 1.0045x over previous
import jax
import jax.numpy as jnp
from jax.experimental import pallas as pl
from jax.experimental.pallas import tpu as pltpu
from jax.experimental.pallas import tpu_sc as plsc

D = 1024
E = 8
N_TOK = 4096
BLK = 128
A_TOT = N_TOK * 2
PAD = A_TOT + E * BLK
NB = PAD // BLK
TBLK = 256
GBLK = N_TOK // TBLK
EPAD = 128
YASG = 2 * N_TOK + TBLK
NW = 32
RPW = PAD // NW
CH = 48
NCH = RPW // CH

_BIG = 1 << 30


def _gate_body(x_ref, w_ref, b_ref, i0_ref, i1_ref, w0_ref, w1_ref,
               c0_ref, c1_ref):
    logits = jnp.dot(x_ref[...], w_ref[...],
                     preferred_element_type=jnp.float32) + b_ref[...]
    m = jnp.max(logits, axis=1, keepdims=True)
    ex = jnp.exp(logits - m)
    p = ex / jnp.sum(ex, axis=1, keepdims=True)
    iota = jax.lax.broadcasted_iota(jnp.int32, (TBLK, EPAD), 1)
    v0 = jnp.max(p, axis=1, keepdims=True)
    i0 = jnp.min(jnp.where(p == v0, iota, _BIG), axis=1, keepdims=True)
    p2 = jnp.where(iota == i0, -1.0, p)
    v1 = jnp.max(p2, axis=1, keepdims=True)
    i1 = jnp.min(jnp.where(p2 == v1, iota, _BIG), axis=1, keepdims=True)
    zeros_i = jnp.zeros((TBLK, EPAD), jnp.int32)
    i0_ref[...] = i0 + zeros_i
    i1_ref[...] = i1 + zeros_i
    zeros_f = jnp.zeros((TBLK, EPAD), jnp.float32)
    w0_ref[...] = v0 + zeros_f
    w1_ref[...] = v1 + zeros_f
    oh0 = (iota == i0).astype(jnp.int32)
    oh1 = (iota == i1).astype(jnp.int32)
    c0_ref[...] = jnp.sum(oh0, axis=0, keepdims=True).reshape(1, 1, EPAD)
    c1_ref[...] = jnp.sum(oh1, axis=0, keepdims=True).reshape(1, 1, EPAD)


def _gating(x, gate_W, gate_b):
    gwp = jnp.zeros((D, EPAD), jnp.float32).at[:, :E].set(gate_W)
    gbp = jnp.full((1, EPAD), -1e30, jnp.float32).at[0, :E].set(gate_b)
    out_sh = [jax.ShapeDtypeStruct((N_TOK, EPAD), jnp.int32),
              jax.ShapeDtypeStruct((N_TOK, EPAD), jnp.int32),
              jax.ShapeDtypeStruct((N_TOK, EPAD), jnp.float32),
              jax.ShapeDtypeStruct((N_TOK, EPAD), jnp.float32),
              jax.ShapeDtypeStruct((GBLK, 1, EPAD), jnp.int32),
              jax.ShapeDtypeStruct((GBLK, 1, EPAD), jnp.int32)]
    blk = pl.BlockSpec((TBLK, EPAD), lambda i: (i, 0))
    cblk = pl.BlockSpec((1, 1, EPAD), lambda i: (i, 0, 0))
    return pl.pallas_call(
        _gate_body,
        grid=(GBLK,),
        in_specs=[pl.BlockSpec((TBLK, D), lambda i: (i, 0)),
                  pl.BlockSpec((D, EPAD), lambda i: (0, 0)),
                  pl.BlockSpec((1, EPAD), lambda i: (0, 0))],
        out_specs=[blk, blk, blk, blk, cblk, cblk],
        out_shape=out_sh,
    )(x, gwp, gbp)


def _router_body(i0_ref, i1_ref, c0_ref, c1_ref, pos_ref, blk_ref, acc_ref):
    step = pl.program_id(0)

    @pl.when(step == 0)
    def _():
        acc_ref[...] = jnp.zeros((8, EPAD), jnp.float32)

    sel = jnp.where(step < GBLK, i0_ref[:, 0:1], i1_ref[:, 0:1])
    lane = jax.lax.broadcasted_iota(jnp.int32, (TBLK, EPAD), 1)
    oh = (lane == sel).astype(jnp.float32)

    tot = (jnp.sum(c0_ref[...], axis=0) +
           jnp.sum(c1_ref[...], axis=0)).astype(jnp.int32)
    pc = (((tot + (BLK - 1)) // BLK) * BLK).astype(jnp.float32)
    iu0 = jax.lax.broadcasted_iota(jnp.int32, (EPAD, EPAD), 0)
    iu1 = jax.lax.broadcasted_iota(jnp.int32, (EPAD, EPAD), 1)
    U = (iu0 < iu1).astype(jnp.float32)
    poff = jnp.dot(pc, U, preferred_element_type=jnp.float32)

    il0 = jax.lax.broadcasted_iota(jnp.int32, (TBLK, TBLK), 0)
    il1 = jax.lax.broadcasted_iota(jnp.int32, (TBLK, TBLK), 1)
    L = (il1 <= il0).astype(jnp.float32)
    csum = jnp.dot(L, oh, preferred_element_type=jnp.float32)

    accrow = acc_ref[0:1, :]
    base = poff + accrow
    posv = jnp.sum(oh * (base + csum - 1.0), axis=1, keepdims=True)
    pos_ref[...] = (posv + jnp.zeros((TBLK, EPAD), jnp.float32)).astype(
        jnp.int32)
    newrow = accrow + jnp.sum(oh, axis=0, keepdims=True)
    acc_ref[...] = jnp.broadcast_to(newrow, (8, EPAD))

    pend = poff + pc
    ib0 = jax.lax.broadcasted_iota(jnp.int32, (EPAD, EPAD), 0)
    ib1 = jax.lax.broadcasted_iota(jnp.int32, (EPAD, EPAD), 1)
    starts = (ib0 * BLK).astype(jnp.float32)
    ge = jnp.where((ib1 < E) & (starts >= pend), 1, 0)
    be = jnp.minimum(jnp.sum(ge, axis=1, keepdims=True), E - 1)
    blk_ref[...] = be + jnp.zeros((EPAD, EPAD), jnp.int32)


def _router(i0b, i1b, cnt0, cnt1):
    cspec = pl.BlockSpec((GBLK, 1, EPAD), lambda i: (0, 0, 0))
    return pl.pallas_call(
        _router_body,
        grid=(2 * GBLK,),
        in_specs=[pl.BlockSpec((TBLK, EPAD), lambda i: (i % GBLK, 0)),
                  pl.BlockSpec((TBLK, EPAD), lambda i: (i % GBLK, 0)),
                  cspec, cspec],
        out_specs=[pl.BlockSpec((TBLK, EPAD), lambda i: (i, 0)),
                   pl.BlockSpec((EPAD, EPAD), lambda i: (0, 0))],
        out_shape=[jax.ShapeDtypeStruct((A_TOT, EPAD), jnp.int32),
                   jax.ShapeDtypeStruct((EPAD, EPAD), jnp.int32)],
        scratch_shapes=[pltpu.VMEM((8, EPAD), jnp.float32)],
    )(i0b, i1b, cnt0, cnt1)


TPW = N_TOK // NW
DCH = 32
DNCH = TPW // DCH


def _sc_dispatch(x, pos0, pos1):
    mesh = plsc.VectorSubcoreMesh(core_axis_name="core",
                                  subcore_axis_name="subcore")

    @pl.kernel(out_type=jax.ShapeDtypeStruct((PAD, D), jnp.float32),
               mesh=mesh,
               scratch_types=[pltpu.VMEM((DCH,), jnp.int32),
                              pltpu.VMEM((DCH,), jnp.int32),
                              pltpu.VMEM((DCH,), jnp.int32),
                              pltpu.VMEM((DCH,), jnp.int32),
                              pltpu.VMEM((DCH, D), jnp.float32),
                              pltpu.VMEM((DCH, D), jnp.float32),
                              pltpu.SemaphoreType.DMA,
                              pltpu.SemaphoreType.DMA,
                              pltpu.SemaphoreType.DMA,
                              pltpu.SemaphoreType.DMA])
    def dispatch_k(x_hbm, p0_hbm, p1_hbm, o_hbm,
                   ia0, ia1, ib0, ib1, buf0, buf1, sa0, sa1, sb0, sb1):
        wid = (jax.lax.axis_index("subcore") * 2
               + jax.lax.axis_index("core"))
        base = wid * TPW
        ia = (ia0, ia1)
        ib = (ib0, ib1)
        buf = (buf0, buf1)
        sa = (sa0, sa1)
        sb = (sb0, sb1)
        cpa = [None, None]
        cpb = [None, None]
        for c in range(DNCH):
            s = c % 2
            if cpa[s] is not None:
                cpa[s].wait()
                cpb[s].wait()
            off = base + c * DCH
            pltpu.sync_copy(x_hbm.at[pl.ds(off, DCH)], buf[s])
            pltpu.sync_copy(p0_hbm.at[pl.ds(off, DCH)], ia[s])
            pltpu.sync_copy(p1_hbm.at[pl.ds(off, DCH)], ib[s])
            cpa[s] = pltpu.async_copy(buf[s], o_hbm.at[ia[s]], sa[s])
            cpb[s] = pltpu.async_copy(buf[s], o_hbm.at[ib[s]], sb[s])
        cpa[0].wait()
        cpb[0].wait()
        cpa[1].wait()
        cpb[1].wait()

    return dispatch_k(x, pos0, pos1)


def _ffn_body(be_ref, x_ref, w1_ref, b1_ref, w2_ref, b2_ref, o_ref):
    h = jnp.dot(x_ref[...], w1_ref[0],
                preferred_element_type=jnp.float32) + b1_ref[0]
    h = jnp.maximum(h, 0.0)
    o_ref[...] = jnp.dot(h, w2_ref[0],
                         preferred_element_type=jnp.float32) + b2_ref[0]


def _grouped_ffn(block_expert, x_sorted, fc1_W, fc1_b, fc2_W, fc2_b):
    spec = pltpu.PrefetchScalarGridSpec(
        num_scalar_prefetch=1,
        grid=(NB,),
        in_specs=[
            pl.BlockSpec((BLK, D), lambda i, be: (i, 0)),
            pl.BlockSpec((1, D, D), lambda i, be: (be[i], 0, 0)),
            pl.BlockSpec((1, 1, D), lambda i, be: (be[i], 0, 0)),
            pl.BlockSpec((1, D, D), lambda i, be: (be[i], 0, 0)),
            pl.BlockSpec((1, 1, D), lambda i, be: (be[i], 0, 0)),
        ],
        out_specs=pl.BlockSpec((BLK, D), lambda i, be: (i, 0)),
    )
    return pl.pallas_call(
        _ffn_body,
        grid_spec=spec,
        out_shape=jax.ShapeDtypeStruct((PAD, D), jnp.float32),
    )(block_expert, x_sorted, fc1_W, fc1_b.reshape(E, 1, D),
      fc2_W, fc2_b.reshape(E, 1, D))


def _sc_scatter(y_sorted, dest):
    mesh = plsc.VectorSubcoreMesh(core_axis_name="core",
                                  subcore_axis_name="subcore")

    @pl.kernel(out_type=jax.ShapeDtypeStruct((YASG, D), jnp.float32),
               mesh=mesh,
               scratch_types=[pltpu.VMEM((CH,), jnp.int32),
                              pltpu.VMEM((CH,), jnp.int32),
                              pltpu.VMEM((CH, D), jnp.float32),
                              pltpu.VMEM((CH, D), jnp.float32),
                              pltpu.SemaphoreType.DMA,
                              pltpu.SemaphoreType.DMA])
    def scatter_k(y_hbm, d_hbm, o_hbm, idx0, idx1, buf0, buf1, sem0, sem1):
        wid = (jax.lax.axis_index("subcore") * 2
               + jax.lax.axis_index("core"))
        base = wid * RPW
        idx = (idx0, idx1)
        buf = (buf0, buf1)
        sem = (sem0, sem1)
        cps = [None, None]
        for c in range(NCH):
            s = c % 2
            if cps[s] is not None:
                cps[s].wait()
            pltpu.sync_copy(y_hbm.at[pl.ds(base + c * CH, CH)], buf[s])
            pltpu.sync_copy(d_hbm.at[pl.ds(base + c * CH, CH)], idx[s])
            cps[s] = pltpu.async_copy(buf[s], o_hbm.at[idx[s]], sem[s])
        cps[0].wait()
        cps[1].wait()

    return scatter_k(y_sorted, dest)


def _combine_body(y0_ref, y1_ref, w0_ref, w1_ref, o_ref):
    o_ref[...] = (w0_ref[:, 0:1] * y0_ref[...] +
                  w1_ref[:, 0:1] * y1_ref[...])


def _combine(yasg, w0b, w1b):
    wblk = pl.BlockSpec((TBLK, EPAD), lambda i: (i, 0))
    return pl.pallas_call(
        _combine_body,
        grid=(GBLK,),
        in_specs=[pl.BlockSpec((TBLK, D), lambda i: (i, 0)),
                  pl.BlockSpec((TBLK, D), lambda i: (i + GBLK, 0)),
                  wblk, wblk],
        out_specs=pl.BlockSpec((TBLK, D), lambda i: (i, 0)),
        out_shape=jax.ShapeDtypeStruct((N_TOK, D), jnp.float32),
    )(yasg, yasg, w0b, w1b)


def kernel(x, gate_W, gate_b, fc1_W, fc1_b, fc2_W, fc2_b):
    i0b, i1b, w0b, w1b, cnt0, cnt1 = _gating(x, gate_W, gate_b)
    pos_all, blkexp = _router(i0b, i1b, cnt0, cnt1)
    pos_flat = pos_all[:, 0]
    block_expert = blkexp[:NB, 0]
    dest = (2 * N_TOK + (jnp.arange(PAD) % TBLK)).astype(jnp.int32)
    dest = dest.at[pos_flat].set(jnp.arange(A_TOT, dtype=jnp.int32))

    x_sorted = _sc_dispatch(x, pos_flat[:N_TOK], pos_flat[N_TOK:])
    y_sorted = _grouped_ffn(block_expert, x_sorted, fc1_W, fc1_b, fc2_W, fc2_b)
    yasg = _sc_scatter(y_sorted, dest)
    return _combine(yasg, w0b, w1b)


# speedup vs baseline: 2.0177x; 1.0017x over previous
import jax
import jax.numpy as jnp
from jax.experimental import pallas as pl
from jax.experimental.pallas import tpu as pltpu
from jax.experimental.pallas import tpu_sc as plsc

D = 1024
E = 8
N_TOK = 4096
BLK = 128
A_TOT = N_TOK * 2
PAD = A_TOT + E * BLK
NB = PAD // BLK
TBLK = 256
GBLK = N_TOK // TBLK
EPAD = 128
YASG = 2 * N_TOK + TBLK
NW = 32
RPW = PAD // NW
CH = 48
NCH = RPW // CH

_BIG = 1 << 30


def _gate_body(x_ref, w_ref, b_ref, i0_ref, i1_ref, w0_ref, w1_ref,
               c0_ref, c1_ref):
    logits = jnp.dot(x_ref[...], w_ref[...],
                     preferred_element_type=jnp.float32) + b_ref[...]
    m = jnp.max(logits, axis=1, keepdims=True)
    ex = jnp.exp(logits - m)
    p = ex / jnp.sum(ex, axis=1, keepdims=True)
    iota = jax.lax.broadcasted_iota(jnp.int32, (TBLK, EPAD), 1)
    v0 = jnp.max(p, axis=1, keepdims=True)
    i0 = jnp.min(jnp.where(p == v0, iota, _BIG), axis=1, keepdims=True)
    p2 = jnp.where(iota == i0, -1.0, p)
    v1 = jnp.max(p2, axis=1, keepdims=True)
    i1 = jnp.min(jnp.where(p2 == v1, iota, _BIG), axis=1, keepdims=True)
    zeros_i = jnp.zeros((TBLK, EPAD), jnp.int32)
    i0_ref[...] = i0 + zeros_i
    i1_ref[...] = i1 + zeros_i
    zeros_f = jnp.zeros((TBLK, EPAD), jnp.float32)
    w0_ref[...] = v0 + zeros_f
    w1_ref[...] = v1 + zeros_f
    oh0 = (iota == i0).astype(jnp.int32)
    oh1 = (iota == i1).astype(jnp.int32)
    c0_ref[...] = jnp.sum(oh0, axis=0, keepdims=True).reshape(1, 1, EPAD)
    c1_ref[...] = jnp.sum(oh1, axis=0, keepdims=True).reshape(1, 1, EPAD)


def _gating(x, gate_W, gate_b):
    gwp = jnp.zeros((D, EPAD), jnp.float32).at[:, :E].set(gate_W)
    gbp = jnp.full((1, EPAD), -1e30, jnp.float32).at[0, :E].set(gate_b)
    out_sh = [jax.ShapeDtypeStruct((N_TOK, EPAD), jnp.int32),
              jax.ShapeDtypeStruct((N_TOK, EPAD), jnp.int32),
              jax.ShapeDtypeStruct((N_TOK, EPAD), jnp.float32),
              jax.ShapeDtypeStruct((N_TOK, EPAD), jnp.float32),
              jax.ShapeDtypeStruct((GBLK, 1, EPAD), jnp.int32),
              jax.ShapeDtypeStruct((GBLK, 1, EPAD), jnp.int32)]
    blk = pl.BlockSpec((TBLK, EPAD), lambda i: (i, 0))
    cblk = pl.BlockSpec((1, 1, EPAD), lambda i: (i, 0, 0))
    return pl.pallas_call(
        _gate_body,
        grid=(GBLK,),
        in_specs=[pl.BlockSpec((TBLK, D), lambda i: (i, 0)),
                  pl.BlockSpec((D, EPAD), lambda i: (0, 0)),
                  pl.BlockSpec((1, EPAD), lambda i: (0, 0))],
        out_specs=[blk, blk, blk, blk, cblk, cblk],
        out_shape=out_sh,
    )(x, gwp, gbp)


def _router_body(i0_ref, i1_ref, c0_ref, c1_ref, pos_ref, blk_ref, acc_ref):
    step = pl.program_id(0)

    @pl.when(step == 0)
    def _():
        acc_ref[...] = jnp.zeros((8, EPAD), jnp.float32)

    sel = jnp.where(step < GBLK, i0_ref[:, 0:1], i1_ref[:, 0:1])
    lane = jax.lax.broadcasted_iota(jnp.int32, (TBLK, EPAD), 1)
    oh = (lane == sel).astype(jnp.float32)

    tot = (jnp.sum(c0_ref[...], axis=0) +
           jnp.sum(c1_ref[...], axis=0)).astype(jnp.int32)
    pc = (((tot + (BLK - 1)) // BLK) * BLK).astype(jnp.float32)
    iu0 = jax.lax.broadcasted_iota(jnp.int32, (EPAD, EPAD), 0)
    iu1 = jax.lax.broadcasted_iota(jnp.int32, (EPAD, EPAD), 1)
    U = (iu0 < iu1).astype(jnp.float32)
    poff = jnp.dot(pc, U, preferred_element_type=jnp.float32)

    il0 = jax.lax.broadcasted_iota(jnp.int32, (TBLK, TBLK), 0)
    il1 = jax.lax.broadcasted_iota(jnp.int32, (TBLK, TBLK), 1)
    L = (il1 <= il0).astype(jnp.float32)
    csum = jnp.dot(L, oh, preferred_element_type=jnp.float32)

    accrow = acc_ref[0:1, :]
    base = poff + accrow
    posv = jnp.sum(oh * (base + csum - 1.0), axis=1, keepdims=True)
    pos_ref[...] = (posv + jnp.zeros((TBLK, EPAD), jnp.float32)).astype(
        jnp.int32)
    newrow = accrow + jnp.sum(oh, axis=0, keepdims=True)
    acc_ref[...] = jnp.broadcast_to(newrow, (8, EPAD))

    pend = poff + pc
    ib0 = jax.lax.broadcasted_iota(jnp.int32, (EPAD, EPAD), 0)
    ib1 = jax.lax.broadcasted_iota(jnp.int32, (EPAD, EPAD), 1)
    starts = (ib0 * BLK).astype(jnp.float32)
    ge = jnp.where((ib1 < E) & (starts >= pend), 1, 0)
    be = jnp.minimum(jnp.sum(ge, axis=1, keepdims=True), E - 1)
    blk_ref[...] = be + jnp.zeros((EPAD, EPAD), jnp.int32)


def _router(i0b, i1b, cnt0, cnt1):
    cspec = pl.BlockSpec((GBLK, 1, EPAD), lambda i: (0, 0, 0))
    return pl.pallas_call(
        _router_body,
        grid=(2 * GBLK,),
        in_specs=[pl.BlockSpec((TBLK, EPAD), lambda i: (i % GBLK, 0)),
                  pl.BlockSpec((TBLK, EPAD), lambda i: (i % GBLK, 0)),
                  cspec, cspec],
        out_specs=[pl.BlockSpec((TBLK, EPAD), lambda i: (i, 0)),
                   pl.BlockSpec((EPAD, EPAD), lambda i: (0, 0))],
        out_shape=[jax.ShapeDtypeStruct((A_TOT, EPAD), jnp.int32),
                   jax.ShapeDtypeStruct((EPAD, EPAD), jnp.int32)],
        scratch_shapes=[pltpu.VMEM((8, EPAD), jnp.float32)],
    )(i0b, i1b, cnt0, cnt1)


TPW = N_TOK // NW
DCH = 32
DNCH = TPW // DCH


def _sc_dispatch(x, pos0, pos1):
    mesh = plsc.VectorSubcoreMesh(core_axis_name="core",
                                  subcore_axis_name="subcore")

    @pl.kernel(out_type=jax.ShapeDtypeStruct((PAD, D), jnp.float32),
               mesh=mesh,
               scratch_types=[pltpu.VMEM((DCH,), jnp.int32),
                              pltpu.VMEM((DCH,), jnp.int32),
                              pltpu.VMEM((DCH,), jnp.int32),
                              pltpu.VMEM((DCH,), jnp.int32),
                              pltpu.VMEM((DCH, D), jnp.float32),
                              pltpu.VMEM((DCH, D), jnp.float32),
                              pltpu.SemaphoreType.DMA,
                              pltpu.SemaphoreType.DMA,
                              pltpu.SemaphoreType.DMA,
                              pltpu.SemaphoreType.DMA])
    def dispatch_k(x_hbm, p0_hbm, p1_hbm, o_hbm,
                   ia0, ia1, ib0, ib1, buf0, buf1, sa0, sa1, sb0, sb1):
        wid = (jax.lax.axis_index("subcore") * 2
               + jax.lax.axis_index("core"))
        base = wid * TPW
        ia = (ia0, ia1)
        ib = (ib0, ib1)
        buf = (buf0, buf1)
        sa = (sa0, sa1)
        sb = (sb0, sb1)
        cpa = [None, None]
        cpb = [None, None]
        for c in range(DNCH):
            s = c % 2
            if cpa[s] is not None:
                cpa[s].wait()
                cpb[s].wait()
            off = base + c * DCH
            pltpu.sync_copy(x_hbm.at[pl.ds(off, DCH)], buf[s])
            pltpu.sync_copy(p0_hbm.at[pl.ds(off, DCH)], ia[s])
            pltpu.sync_copy(p1_hbm.at[pl.ds(off, DCH)], ib[s])
            cpa[s] = pltpu.async_copy(buf[s], o_hbm.at[ia[s]], sa[s])
            cpb[s] = pltpu.async_copy(buf[s], o_hbm.at[ib[s]], sb[s])
        cpa[0].wait()
        cpb[0].wait()
        cpa[1].wait()
        cpb[1].wait()

    return dispatch_k(x, pos0, pos1)


def _ffn_body(be_ref, x_ref, w1_ref, b1_ref, w2_ref, b2_ref, o_ref):
    h = jnp.dot(x_ref[...], w1_ref[0],
                preferred_element_type=jnp.float32,
                precision=jax.lax.Precision.DEFAULT) + b1_ref[0]
    h = jnp.maximum(h, 0.0)
    o_ref[...] = jnp.dot(h, w2_ref[0],
                         preferred_element_type=jnp.float32,
                         precision=jax.lax.Precision.DEFAULT) + b2_ref[0]


def _grouped_ffn(block_expert, x_sorted, fc1_W, fc1_b, fc2_W, fc2_b):
    spec = pltpu.PrefetchScalarGridSpec(
        num_scalar_prefetch=1,
        grid=(NB,),
        in_specs=[
            pl.BlockSpec((BLK, D), lambda i, be: (i, 0)),
            pl.BlockSpec((1, D, D), lambda i, be: (be[i], 0, 0)),
            pl.BlockSpec((1, 1, D), lambda i, be: (be[i], 0, 0)),
            pl.BlockSpec((1, D, D), lambda i, be: (be[i], 0, 0)),
            pl.BlockSpec((1, 1, D), lambda i, be: (be[i], 0, 0)),
        ],
        out_specs=pl.BlockSpec((BLK, D), lambda i, be: (i, 0)),
    )
    return pl.pallas_call(
        _ffn_body,
        grid_spec=spec,
        out_shape=jax.ShapeDtypeStruct((PAD, D), jnp.float32),
    )(block_expert, x_sorted, fc1_W, fc1_b.reshape(E, 1, D),
      fc2_W, fc2_b.reshape(E, 1, D))


def _sc_scatter(y_sorted, dest):
    mesh = plsc.VectorSubcoreMesh(core_axis_name="core",
                                  subcore_axis_name="subcore")

    @pl.kernel(out_type=jax.ShapeDtypeStruct((YASG, D), jnp.float32),
               mesh=mesh,
               scratch_types=[pltpu.VMEM((CH,), jnp.int32),
                              pltpu.VMEM((CH,), jnp.int32),
                              pltpu.VMEM((CH, D), jnp.float32),
                              pltpu.VMEM((CH, D), jnp.float32),
                              pltpu.SemaphoreType.DMA,
                              pltpu.SemaphoreType.DMA])
    def scatter_k(y_hbm, d_hbm, o_hbm, idx0, idx1, buf0, buf1, sem0, sem1):
        wid = (jax.lax.axis_index("subcore") * 2
               + jax.lax.axis_index("core"))
        base = wid * RPW
        idx = (idx0, idx1)
        buf = (buf0, buf1)
        sem = (sem0, sem1)
        cps = [None, None]
        for c in range(NCH):
            s = c % 2
            if cps[s] is not None:
                cps[s].wait()
            pltpu.sync_copy(y_hbm.at[pl.ds(base + c * CH, CH)], buf[s])
            pltpu.sync_copy(d_hbm.at[pl.ds(base + c * CH, CH)], idx[s])
            cps[s] = pltpu.async_copy(buf[s], o_hbm.at[idx[s]], sem[s])
        cps[0].wait()
        cps[1].wait()

    return scatter_k(y_sorted, dest)


def _combine_body(y0_ref, y1_ref, w0_ref, w1_ref, o_ref):
    o_ref[...] = (w0_ref[:, 0:1] * y0_ref[...] +
                  w1_ref[:, 0:1] * y1_ref[...])


def _combine(yasg, w0b, w1b):
    wblk = pl.BlockSpec((TBLK, EPAD), lambda i: (i, 0))
    return pl.pallas_call(
        _combine_body,
        grid=(GBLK,),
        in_specs=[pl.BlockSpec((TBLK, D), lambda i: (i, 0)),
                  pl.BlockSpec((TBLK, D), lambda i: (i + GBLK, 0)),
                  wblk, wblk],
        out_specs=pl.BlockSpec((TBLK, D), lambda i: (i, 0)),
        out_shape=jax.ShapeDtypeStruct((N_TOK, D), jnp.float32),
    )(yasg, yasg, w0b, w1b)


def kernel(x, gate_W, gate_b, fc1_W, fc1_b, fc2_W, fc2_b):
    i0b, i1b, w0b, w1b, cnt0, cnt1 = _gating(x, gate_W, gate_b)
    pos_all, blkexp = _router(i0b, i1b, cnt0, cnt1)
    pos_flat = pos_all[:, 0]
    block_expert = blkexp[:NB, 0]
    dest = (2 * N_TOK + (jnp.arange(PAD) % TBLK)).astype(jnp.int32)
    dest = dest.at[pos_flat].set(jnp.arange(A_TOT, dtype=jnp.int32))

    x_sorted = _sc_dispatch(x, pos_flat[:N_TOK], pos_flat[N_TOK:])
    y_sorted = _grouped_ffn(block_expert, x_sorted, fc1_W, fc1_b, fc2_W, fc2_b)
    yasg = _sc_scatter(y_sorted, dest)
    return _combine(yasg, w0b, w1b)


# speedup vs baseline: 2.0954x; 1.0385x over previous
import jax
import jax.numpy as jnp
from jax.experimental import pallas as pl
from jax.experimental.pallas import tpu as pltpu
from jax.experimental.pallas import tpu_sc as plsc

D = 1024
E = 8
N_TOK = 4096
BLK = 256
A_TOT = N_TOK * 2
PAD = A_TOT + E * BLK
NB = PAD // BLK
TBLK = 256
GBLK = N_TOK // TBLK
EPAD = 128
YASG = 2 * N_TOK + TBLK
NW = 32
RPW = PAD // NW
CH = 40
NCH = RPW // CH

_BIG = 1 << 30


def _gate_body(x_ref, w_ref, b_ref, i0_ref, i1_ref, w0_ref, w1_ref,
               c0_ref, c1_ref):
    logits = jnp.dot(x_ref[...], w_ref[...],
                     preferred_element_type=jnp.float32) + b_ref[...]
    m = jnp.max(logits, axis=1, keepdims=True)
    ex = jnp.exp(logits - m)
    p = ex / jnp.sum(ex, axis=1, keepdims=True)
    iota = jax.lax.broadcasted_iota(jnp.int32, (TBLK, EPAD), 1)
    v0 = jnp.max(p, axis=1, keepdims=True)
    i0 = jnp.min(jnp.where(p == v0, iota, _BIG), axis=1, keepdims=True)
    p2 = jnp.where(iota == i0, -1.0, p)
    v1 = jnp.max(p2, axis=1, keepdims=True)
    i1 = jnp.min(jnp.where(p2 == v1, iota, _BIG), axis=1, keepdims=True)
    zeros_i = jnp.zeros((TBLK, EPAD), jnp.int32)
    i0_ref[...] = i0 + zeros_i
    i1_ref[...] = i1 + zeros_i
    zeros_f = jnp.zeros((TBLK, EPAD), jnp.float32)
    w0_ref[...] = v0 + zeros_f
    w1_ref[...] = v1 + zeros_f
    oh0 = (iota == i0).astype(jnp.int32)
    oh1 = (iota == i1).astype(jnp.int32)
    c0_ref[...] = jnp.sum(oh0, axis=0, keepdims=True).reshape(1, 1, EPAD)
    c1_ref[...] = jnp.sum(oh1, axis=0, keepdims=True).reshape(1, 1, EPAD)


def _gating(x, gate_W, gate_b):
    gwp = jnp.zeros((D, EPAD), jnp.float32).at[:, :E].set(gate_W)
    gbp = jnp.full((1, EPAD), -1e30, jnp.float32).at[0, :E].set(gate_b)
    out_sh = [jax.ShapeDtypeStruct((N_TOK, EPAD), jnp.int32),
              jax.ShapeDtypeStruct((N_TOK, EPAD), jnp.int32),
              jax.ShapeDtypeStruct((N_TOK, EPAD), jnp.float32),
              jax.ShapeDtypeStruct((N_TOK, EPAD), jnp.float32),
              jax.ShapeDtypeStruct((GBLK, 1, EPAD), jnp.int32),
              jax.ShapeDtypeStruct((GBLK, 1, EPAD), jnp.int32)]
    blk = pl.BlockSpec((TBLK, EPAD), lambda i: (i, 0))
    cblk = pl.BlockSpec((1, 1, EPAD), lambda i: (i, 0, 0))
    return pl.pallas_call(
        _gate_body,
        grid=(GBLK,),
        in_specs=[pl.BlockSpec((TBLK, D), lambda i: (i, 0)),
                  pl.BlockSpec((D, EPAD), lambda i: (0, 0)),
                  pl.BlockSpec((1, EPAD), lambda i: (0, 0))],
        out_specs=[blk, blk, blk, blk, cblk, cblk],
        out_shape=out_sh,
    )(x, gwp, gbp)


def _router_body(i0_ref, i1_ref, c0_ref, c1_ref, pos_ref, blk_ref, acc_ref):
    step = pl.program_id(0)

    @pl.when(step == 0)
    def _():
        acc_ref[...] = jnp.zeros((8, EPAD), jnp.float32)

    sel = jnp.where(step < GBLK, i0_ref[:, 0:1], i1_ref[:, 0:1])
    lane = jax.lax.broadcasted_iota(jnp.int32, (TBLK, EPAD), 1)
    oh = (lane == sel).astype(jnp.float32)

    tot = (jnp.sum(c0_ref[...], axis=0) +
           jnp.sum(c1_ref[...], axis=0)).astype(jnp.int32)
    pc = (((tot + (BLK - 1)) // BLK) * BLK).astype(jnp.float32)
    iu0 = jax.lax.broadcasted_iota(jnp.int32, (EPAD, EPAD), 0)
    iu1 = jax.lax.broadcasted_iota(jnp.int32, (EPAD, EPAD), 1)
    U = (iu0 < iu1).astype(jnp.float32)
    poff = jnp.dot(pc, U, preferred_element_type=jnp.float32)

    il0 = jax.lax.broadcasted_iota(jnp.int32, (TBLK, TBLK), 0)
    il1 = jax.lax.broadcasted_iota(jnp.int32, (TBLK, TBLK), 1)
    L = (il1 <= il0).astype(jnp.float32)
    csum = jnp.dot(L, oh, preferred_element_type=jnp.float32)

    accrow = acc_ref[0:1, :]
    base = poff + accrow
    posv = jnp.sum(oh * (base + csum - 1.0), axis=1, keepdims=True)
    pos_ref[...] = (posv + jnp.zeros((TBLK, EPAD), jnp.float32)).astype(
        jnp.int32)
    newrow = accrow + jnp.sum(oh, axis=0, keepdims=True)
    acc_ref[...] = jnp.broadcast_to(newrow, (8, EPAD))

    pend = poff + pc
    ib0 = jax.lax.broadcasted_iota(jnp.int32, (EPAD, EPAD), 0)
    ib1 = jax.lax.broadcasted_iota(jnp.int32, (EPAD, EPAD), 1)
    starts = (ib0 * BLK).astype(jnp.float32)
    ge = jnp.where((ib1 < E) & (starts >= pend), 1, 0)
    be = jnp.minimum(jnp.sum(ge, axis=1, keepdims=True), E - 1)
    blk_ref[...] = be + jnp.zeros((EPAD, EPAD), jnp.int32)


def _router(i0b, i1b, cnt0, cnt1):
    cspec = pl.BlockSpec((GBLK, 1, EPAD), lambda i: (0, 0, 0))
    return pl.pallas_call(
        _router_body,
        grid=(2 * GBLK,),
        in_specs=[pl.BlockSpec((TBLK, EPAD), lambda i: (i % GBLK, 0)),
                  pl.BlockSpec((TBLK, EPAD), lambda i: (i % GBLK, 0)),
                  cspec, cspec],
        out_specs=[pl.BlockSpec((TBLK, EPAD), lambda i: (i, 0)),
                   pl.BlockSpec((EPAD, EPAD), lambda i: (0, 0))],
        out_shape=[jax.ShapeDtypeStruct((A_TOT, EPAD), jnp.int32),
                   jax.ShapeDtypeStruct((EPAD, EPAD), jnp.int32)],
        scratch_shapes=[pltpu.VMEM((8, EPAD), jnp.float32)],
    )(i0b, i1b, cnt0, cnt1)


TPW = N_TOK // NW
DCH = 32
DNCH = TPW // DCH


def _sc_dispatch(x, pos0, pos1):
    mesh = plsc.VectorSubcoreMesh(core_axis_name="core",
                                  subcore_axis_name="subcore")

    @pl.kernel(out_type=jax.ShapeDtypeStruct((PAD, D), jnp.float32),
               mesh=mesh,
               scratch_types=[pltpu.VMEM((DCH,), jnp.int32),
                              pltpu.VMEM((DCH,), jnp.int32),
                              pltpu.VMEM((DCH,), jnp.int32),
                              pltpu.VMEM((DCH,), jnp.int32),
                              pltpu.VMEM((DCH, D), jnp.float32),
                              pltpu.VMEM((DCH, D), jnp.float32),
                              pltpu.SemaphoreType.DMA,
                              pltpu.SemaphoreType.DMA,
                              pltpu.SemaphoreType.DMA,
                              pltpu.SemaphoreType.DMA])
    def dispatch_k(x_hbm, p0_hbm, p1_hbm, o_hbm,
                   ia0, ia1, ib0, ib1, buf0, buf1, sa0, sa1, sb0, sb1):
        wid = (jax.lax.axis_index("subcore") * 2
               + jax.lax.axis_index("core"))
        base = wid * TPW
        ia = (ia0, ia1)
        ib = (ib0, ib1)
        buf = (buf0, buf1)
        sa = (sa0, sa1)
        sb = (sb0, sb1)
        cpa = [None, None]
        cpb = [None, None]
        for c in range(DNCH):
            s = c % 2
            if cpa[s] is not None:
                cpa[s].wait()
                cpb[s].wait()
            off = base + c * DCH
            pltpu.sync_copy(x_hbm.at[pl.ds(off, DCH)], buf[s])
            pltpu.sync_copy(p0_hbm.at[pl.ds(off, DCH)], ia[s])
            pltpu.sync_copy(p1_hbm.at[pl.ds(off, DCH)], ib[s])
            cpa[s] = pltpu.async_copy(buf[s], o_hbm.at[ia[s]], sa[s])
            cpb[s] = pltpu.async_copy(buf[s], o_hbm.at[ib[s]], sb[s])
        cpa[0].wait()
        cpb[0].wait()
        cpa[1].wait()
        cpb[1].wait()

    return dispatch_k(x, pos0, pos1)


def _ffn_body(be_ref, x_ref, w1_ref, b1_ref, w2_ref, b2_ref, o_ref):
    h = jnp.dot(x_ref[...], w1_ref[0],
                preferred_element_type=jnp.float32,
                precision=jax.lax.Precision.DEFAULT) + b1_ref[0]
    h = jnp.maximum(h, 0.0)
    o_ref[...] = jnp.dot(h, w2_ref[0],
                         preferred_element_type=jnp.float32,
                         precision=jax.lax.Precision.DEFAULT) + b2_ref[0]


def _grouped_ffn(block_expert, x_sorted, fc1_W, fc1_b, fc2_W, fc2_b):
    spec = pltpu.PrefetchScalarGridSpec(
        num_scalar_prefetch=1,
        grid=(NB,),
        in_specs=[
            pl.BlockSpec((BLK, D), lambda i, be: (i, 0)),
            pl.BlockSpec((1, D, D), lambda i, be: (be[i], 0, 0)),
            pl.BlockSpec((1, 1, D), lambda i, be: (be[i], 0, 0)),
            pl.BlockSpec((1, D, D), lambda i, be: (be[i], 0, 0)),
            pl.BlockSpec((1, 1, D), lambda i, be: (be[i], 0, 0)),
        ],
        out_specs=pl.BlockSpec((BLK, D), lambda i, be: (i, 0)),
    )
    return pl.pallas_call(
        _ffn_body,
        grid_spec=spec,
        out_shape=jax.ShapeDtypeStruct((PAD, D), jnp.float32),
    )(block_expert, x_sorted, fc1_W, fc1_b.reshape(E, 1, D),
      fc2_W, fc2_b.reshape(E, 1, D))


def _sc_scatter(y_sorted, dest):
    mesh = plsc.VectorSubcoreMesh(core_axis_name="core",
                                  subcore_axis_name="subcore")

    @pl.kernel(out_type=jax.ShapeDtypeStruct((YASG, D), jnp.float32),
               mesh=mesh,
               scratch_types=[pltpu.VMEM((CH,), jnp.int32),
                              pltpu.VMEM((CH,), jnp.int32),
                              pltpu.VMEM((CH, D), jnp.float32),
                              pltpu.VMEM((CH, D), jnp.float32),
                              pltpu.SemaphoreType.DMA,
                              pltpu.SemaphoreType.DMA])
    def scatter_k(y_hbm, d_hbm, o_hbm, idx0, idx1, buf0, buf1, sem0, sem1):
        wid = (jax.lax.axis_index("subcore") * 2
               + jax.lax.axis_index("core"))
        base = wid * RPW
        idx = (idx0, idx1)
        buf = (buf0, buf1)
        sem = (sem0, sem1)
        cps = [None, None]
        for c in range(NCH):
            s = c % 2
            if cps[s] is not None:
                cps[s].wait()
            pltpu.sync_copy(y_hbm.at[pl.ds(base + c * CH, CH)], buf[s])
            pltpu.sync_copy(d_hbm.at[pl.ds(base + c * CH, CH)], idx[s])
            cps[s] = pltpu.async_copy(buf[s], o_hbm.at[idx[s]], sem[s])
        cps[0].wait()
        cps[1].wait()

    return scatter_k(y_sorted, dest)


def _combine_body(y0_ref, y1_ref, w0_ref, w1_ref, o_ref):
    o_ref[...] = (w0_ref[:, 0:1] * y0_ref[...] +
                  w1_ref[:, 0:1] * y1_ref[...])


def _combine(yasg, w0b, w1b):
    wblk = pl.BlockSpec((TBLK, EPAD), lambda i: (i, 0))
    return pl.pallas_call(
        _combine_body,
        grid=(GBLK,),
        in_specs=[pl.BlockSpec((TBLK, D), lambda i: (i, 0)),
                  pl.BlockSpec((TBLK, D), lambda i: (i + GBLK, 0)),
                  wblk, wblk],
        out_specs=pl.BlockSpec((TBLK, D), lambda i: (i, 0)),
        out_shape=jax.ShapeDtypeStruct((N_TOK, D), jnp.float32),
    )(yasg, yasg, w0b, w1b)


def kernel(x, gate_W, gate_b, fc1_W, fc1_b, fc2_W, fc2_b):
    i0b, i1b, w0b, w1b, cnt0, cnt1 = _gating(x, gate_W, gate_b)
    pos_all, blkexp = _router(i0b, i1b, cnt0, cnt1)
    pos_flat = pos_all[:, 0]
    block_expert = blkexp[:NB, 0]
    dest = (2 * N_TOK + (jnp.arange(PAD) % TBLK)).astype(jnp.int32)
    dest = dest.at[pos_flat].set(jnp.arange(A_TOT, dtype=jnp.int32))

    x_sorted = _sc_dispatch(x, pos_flat[:N_TOK], pos_flat[N_TOK:])
    y_sorted = _grouped_ffn(block_expert, x_sorted, fc1_W, fc1_b, fc2_W, fc2_b)
    yasg = _sc_scatter(y_sorted, dest)
    return _combine(yasg, w0b, w1b)


# speedup vs baseline: 2.2605x; 1.0788x over previous
import jax
import jax.numpy as jnp
from jax.experimental import pallas as pl
from jax.experimental.pallas import tpu as pltpu
from jax.experimental.pallas import tpu_sc as plsc

D = 1024
E = 8
N_TOK = 4096
BLK = 256
A_TOT = N_TOK * 2
PAD = A_TOT + E * BLK
NB = PAD // BLK
TBLK = 512
GBLK = N_TOK // TBLK
EPAD = 128
YASG = 2 * N_TOK + TBLK
NW = 32
RPW = PAD // NW
CH = 40
NCH = RPW // CH

_BIG = 1 << 30


def _gate_body(x_ref, w_ref, b_ref, i0_ref, i1_ref, w0_ref, w1_ref,
               c0_ref, c1_ref):
    logits = jnp.dot(x_ref[...], w_ref[...],
                     preferred_element_type=jnp.float32) + b_ref[...]
    m = jnp.max(logits, axis=1, keepdims=True)
    ex = jnp.exp(logits - m)
    p = ex / jnp.sum(ex, axis=1, keepdims=True)
    iota = jax.lax.broadcasted_iota(jnp.int32, (TBLK, EPAD), 1)
    v0 = jnp.max(p, axis=1, keepdims=True)
    i0 = jnp.min(jnp.where(p == v0, iota, _BIG), axis=1, keepdims=True)
    p2 = jnp.where(iota == i0, -1.0, p)
    v1 = jnp.max(p2, axis=1, keepdims=True)
    i1 = jnp.min(jnp.where(p2 == v1, iota, _BIG), axis=1, keepdims=True)
    zeros_i = jnp.zeros((TBLK, EPAD), jnp.int32)
    i0_ref[...] = i0 + zeros_i
    i1_ref[...] = i1 + zeros_i
    zeros_f = jnp.zeros((TBLK, EPAD), jnp.float32)
    w0_ref[...] = v0 + zeros_f
    w1_ref[...] = v1 + zeros_f
    oh0 = (iota == i0).astype(jnp.int32)
    oh1 = (iota == i1).astype(jnp.int32)
    c0_ref[...] = jnp.sum(oh0, axis=0, keepdims=True).reshape(1, 1, EPAD)
    c1_ref[...] = jnp.sum(oh1, axis=0, keepdims=True).reshape(1, 1, EPAD)


def _gating(x, gate_W, gate_b):
    gwp = jnp.zeros((D, EPAD), jnp.float32).at[:, :E].set(gate_W)
    gbp = jnp.full((1, EPAD), -1e30, jnp.float32).at[0, :E].set(gate_b)
    out_sh = [jax.ShapeDtypeStruct((N_TOK, EPAD), jnp.int32),
              jax.ShapeDtypeStruct((N_TOK, EPAD), jnp.int32),
              jax.ShapeDtypeStruct((N_TOK, EPAD), jnp.float32),
              jax.ShapeDtypeStruct((N_TOK, EPAD), jnp.float32),
              jax.ShapeDtypeStruct((GBLK, 1, EPAD), jnp.int32),
              jax.ShapeDtypeStruct((GBLK, 1, EPAD), jnp.int32)]
    blk = pl.BlockSpec((TBLK, EPAD), lambda i: (i, 0))
    cblk = pl.BlockSpec((1, 1, EPAD), lambda i: (i, 0, 0))
    return pl.pallas_call(
        _gate_body,
        grid=(GBLK,),
        in_specs=[pl.BlockSpec((TBLK, D), lambda i: (i, 0)),
                  pl.BlockSpec((D, EPAD), lambda i: (0, 0)),
                  pl.BlockSpec((1, EPAD), lambda i: (0, 0))],
        out_specs=[blk, blk, blk, blk, cblk, cblk],
        out_shape=out_sh,
    )(x, gwp, gbp)


def _router_body(i0_ref, i1_ref, c0_ref, c1_ref, pos_ref, blk_ref, acc_ref):
    step = pl.program_id(0)

    @pl.when(step == 0)
    def _():
        acc_ref[...] = jnp.zeros((8, EPAD), jnp.float32)

    sel = jnp.where(step < GBLK, i0_ref[:, 0:1], i1_ref[:, 0:1])
    lane = jax.lax.broadcasted_iota(jnp.int32, (TBLK, EPAD), 1)
    oh = (lane == sel).astype(jnp.float32)

    tot = (jnp.sum(c0_ref[...], axis=0) +
           jnp.sum(c1_ref[...], axis=0)).astype(jnp.int32)
    pc = (((tot + (BLK - 1)) // BLK) * BLK).astype(jnp.float32)
    iu0 = jax.lax.broadcasted_iota(jnp.int32, (EPAD, EPAD), 0)
    iu1 = jax.lax.broadcasted_iota(jnp.int32, (EPAD, EPAD), 1)
    U = (iu0 < iu1).astype(jnp.float32)
    poff = jnp.dot(pc, U, preferred_element_type=jnp.float32)

    il0 = jax.lax.broadcasted_iota(jnp.int32, (TBLK, TBLK), 0)
    il1 = jax.lax.broadcasted_iota(jnp.int32, (TBLK, TBLK), 1)
    L = (il1 <= il0).astype(jnp.float32)
    csum = jnp.dot(L, oh, preferred_element_type=jnp.float32)

    accrow = acc_ref[0:1, :]
    base = poff + accrow
    posv = jnp.sum(oh * (base + csum - 1.0), axis=1, keepdims=True)
    pos_ref[...] = (posv + jnp.zeros((TBLK, EPAD), jnp.float32)).astype(
        jnp.int32)
    newrow = accrow + jnp.sum(oh, axis=0, keepdims=True)
    acc_ref[...] = jnp.broadcast_to(newrow, (8, EPAD))

    pend = poff + pc
    ib0 = jax.lax.broadcasted_iota(jnp.int32, (EPAD, EPAD), 0)
    ib1 = jax.lax.broadcasted_iota(jnp.int32, (EPAD, EPAD), 1)
    starts = (ib0 * BLK).astype(jnp.float32)
    ge = jnp.where((ib1 < E) & (starts >= pend), 1, 0)
    be = jnp.minimum(jnp.sum(ge, axis=1, keepdims=True), E - 1)
    blk_ref[...] = be + jnp.zeros((EPAD, EPAD), jnp.int32)


def _router(i0b, i1b, cnt0, cnt1):
    cspec = pl.BlockSpec((GBLK, 1, EPAD), lambda i: (0, 0, 0))
    return pl.pallas_call(
        _router_body,
        grid=(2 * GBLK,),
        in_specs=[pl.BlockSpec((TBLK, EPAD), lambda i: (i % GBLK, 0)),
                  pl.BlockSpec((TBLK, EPAD), lambda i: (i % GBLK, 0)),
                  cspec, cspec],
        out_specs=[pl.BlockSpec((TBLK, EPAD), lambda i: (i, 0)),
                   pl.BlockSpec((EPAD, EPAD), lambda i: (0, 0))],
        out_shape=[jax.ShapeDtypeStruct((A_TOT, EPAD), jnp.int32),
                   jax.ShapeDtypeStruct((EPAD, EPAD), jnp.int32)],
        scratch_shapes=[pltpu.VMEM((8, EPAD), jnp.float32)],
    )(i0b, i1b, cnt0, cnt1)


TPW = N_TOK // NW
DCH = 32
DNCH = TPW // DCH


def _sc_dispatch(x, pos0, pos1):
    mesh = plsc.VectorSubcoreMesh(core_axis_name="core",
                                  subcore_axis_name="subcore")

    @pl.kernel(out_type=jax.ShapeDtypeStruct((PAD, D), jnp.float32),
               mesh=mesh,
               scratch_types=[pltpu.VMEM((DCH,), jnp.int32),
                              pltpu.VMEM((DCH,), jnp.int32),
                              pltpu.VMEM((DCH,), jnp.int32),
                              pltpu.VMEM((DCH,), jnp.int32),
                              pltpu.VMEM((DCH, D), jnp.float32),
                              pltpu.VMEM((DCH, D), jnp.float32),
                              pltpu.SemaphoreType.DMA,
                              pltpu.SemaphoreType.DMA,
                              pltpu.SemaphoreType.DMA,
                              pltpu.SemaphoreType.DMA])
    def dispatch_k(x_hbm, p0_hbm, p1_hbm, o_hbm,
                   ia0, ia1, ib0, ib1, buf0, buf1, sa0, sa1, sb0, sb1):
        wid = (jax.lax.axis_index("subcore") * 2
               + jax.lax.axis_index("core"))
        base = wid * TPW
        ia = (ia0, ia1)
        ib = (ib0, ib1)
        buf = (buf0, buf1)
        sa = (sa0, sa1)
        sb = (sb0, sb1)
        cpa = [None, None]
        cpb = [None, None]
        for c in range(DNCH):
            s = c % 2
            if cpa[s] is not None:
                cpa[s].wait()
                cpb[s].wait()
            off = base + c * DCH
            pltpu.sync_copy(x_hbm.at[pl.ds(off, DCH)], buf[s])
            pltpu.sync_copy(p0_hbm.at[pl.ds(off, DCH)], ia[s])
            pltpu.sync_copy(p1_hbm.at[pl.ds(off, DCH)], ib[s])
            cpa[s] = pltpu.async_copy(buf[s], o_hbm.at[ia[s]], sa[s])
            cpb[s] = pltpu.async_copy(buf[s], o_hbm.at[ib[s]], sb[s])
        cpa[0].wait()
        cpb[0].wait()
        cpa[1].wait()
        cpb[1].wait()

    return dispatch_k(x, pos0, pos1)


def _ffn_body(be_ref, x_ref, w1_ref, b1_ref, w2_ref, b2_ref, o_ref):
    h = jnp.dot(x_ref[...], w1_ref[0],
                preferred_element_type=jnp.float32,
                precision=jax.lax.Precision.DEFAULT) + b1_ref[0]
    h = jnp.maximum(h, 0.0)
    o_ref[...] = jnp.dot(h, w2_ref[0],
                         preferred_element_type=jnp.float32,
                         precision=jax.lax.Precision.DEFAULT) + b2_ref[0]


def _grouped_ffn(block_expert, x_sorted, fc1_W, fc1_b, fc2_W, fc2_b):
    spec = pltpu.PrefetchScalarGridSpec(
        num_scalar_prefetch=1,
        grid=(NB,),
        in_specs=[
            pl.BlockSpec((BLK, D), lambda i, be: (i, 0)),
            pl.BlockSpec((1, D, D), lambda i, be: (be[i], 0, 0)),
            pl.BlockSpec((1, 1, D), lambda i, be: (be[i], 0, 0)),
            pl.BlockSpec((1, D, D), lambda i, be: (be[i], 0, 0)),
            pl.BlockSpec((1, 1, D), lambda i, be: (be[i], 0, 0)),
        ],
        out_specs=pl.BlockSpec((BLK, D), lambda i, be: (i, 0)),
    )
    return pl.pallas_call(
        _ffn_body,
        grid_spec=spec,
        out_shape=jax.ShapeDtypeStruct((PAD, D), jnp.float32),
    )(block_expert, x_sorted, fc1_W, fc1_b.reshape(E, 1, D),
      fc2_W, fc2_b.reshape(E, 1, D))


def _sc_scatter(y_sorted, dest):
    mesh = plsc.VectorSubcoreMesh(core_axis_name="core",
                                  subcore_axis_name="subcore")

    @pl.kernel(out_type=jax.ShapeDtypeStruct((YASG, D), jnp.float32),
               mesh=mesh,
               scratch_types=[pltpu.VMEM((CH,), jnp.int32),
                              pltpu.VMEM((CH,), jnp.int32),
                              pltpu.VMEM((CH, D), jnp.float32),
                              pltpu.VMEM((CH, D), jnp.float32),
                              pltpu.SemaphoreType.DMA,
                              pltpu.SemaphoreType.DMA])
    def scatter_k(y_hbm, d_hbm, o_hbm, idx0, idx1, buf0, buf1, sem0, sem1):
        wid = (jax.lax.axis_index("subcore") * 2
               + jax.lax.axis_index("core"))
        base = wid * RPW
        idx = (idx0, idx1)
        buf = (buf0, buf1)
        sem = (sem0, sem1)
        cps = [None, None]
        for c in range(NCH):
            s = c % 2
            if cps[s] is not None:
                cps[s].wait()
            pltpu.sync_copy(y_hbm.at[pl.ds(base + c * CH, CH)], buf[s])
            pltpu.sync_copy(d_hbm.at[pl.ds(base + c * CH, CH)], idx[s])
            cps[s] = pltpu.async_copy(buf[s], o_hbm.at[idx[s]], sem[s])
        cps[0].wait()
        cps[1].wait()

    return scatter_k(y_sorted, dest)


def _combine_body(y0_ref, y1_ref, w0_ref, w1_ref, o_ref):
    o_ref[...] = (w0_ref[:, 0:1] * y0_ref[...] +
                  w1_ref[:, 0:1] * y1_ref[...])


def _combine(yasg, w0b, w1b):
    wblk = pl.BlockSpec((TBLK, EPAD), lambda i: (i, 0))
    return pl.pallas_call(
        _combine_body,
        grid=(GBLK,),
        in_specs=[pl.BlockSpec((TBLK, D), lambda i: (i, 0)),
                  pl.BlockSpec((TBLK, D), lambda i: (i + GBLK, 0)),
                  wblk, wblk],
        out_specs=pl.BlockSpec((TBLK, D), lambda i: (i, 0)),
        out_shape=jax.ShapeDtypeStruct((N_TOK, D), jnp.float32),
    )(yasg, yasg, w0b, w1b)


def kernel(x, gate_W, gate_b, fc1_W, fc1_b, fc2_W, fc2_b):
    i0b, i1b, w0b, w1b, cnt0, cnt1 = _gating(x, gate_W, gate_b)
    pos_all, blkexp = _router(i0b, i1b, cnt0, cnt1)
    pos_flat = pos_all[:, 0]
    block_expert = blkexp[:NB, 0]

    x_sorted = _sc_dispatch(x, pos_flat[:N_TOK], pos_flat[N_TOK:])
    y_sorted = _grouped_ffn(block_expert, x_sorted, fc1_W, fc1_b, fc2_W, fc2_b)
    dest = (2 * N_TOK + (jnp.arange(PAD) % TBLK)).astype(jnp.int32)
    dest = dest.at[pos_flat].set(jnp.arange(A_TOT, dtype=jnp.int32))
    yasg = _sc_scatter(y_sorted, dest)
    return _combine(yasg, w0b, w1b)
